# baseline (speedup 1.0000x reference)
import functools
import math

import jax
import jax.numpy as jnp
from jax import lax
from jax.experimental import pallas as pl
from jax.experimental.pallas import tpu as pltpu
from jax.experimental.pallas import tpu_sc as plsc

D_MODEL = 64
B_TOTAL = 4096 * 200
NUM_CORES = 2
NUM_SUBCORES = 16
NUM_WORKERS = NUM_CORES * NUM_SUBCORES
BPW = B_TOTAL // NUM_WORKERS
CHUNK = 512
NCHUNKS = BPW // CHUNK
IDX_SUB = 128
NSTREAMS = CHUNK // IDX_SUB
SCALE = math.sqrt(D_MODEL)

_mesh = plsc.VectorSubcoreMesh(core_axis_name="c", subcore_axis_name="s")


@functools.partial(
    pl.kernel,
    mesh=_mesh,
    out_type=jax.ShapeDtypeStruct((B_TOTAL, D_MODEL), jnp.float32),
    scratch_types=[
        pltpu.VMEM((CHUNK,), jnp.int32),
        pltpu.VMEM((CHUNK, D_MODEL), jnp.float32),
        pltpu.SemaphoreType.DMA,
    ],
    compiler_params=pltpu.CompilerParams(use_tc_tiling_on_sc=False),
)
def _embed_sc(x_hbm, lut_hbm, out_hbm, idx_v, rows_v, gsem):
    wid = lax.axis_index("s") * NUM_CORES + lax.axis_index("c")
    wbase = wid * BPW

    def chunk_body(c, carry):
        base = wbase + c * CHUNK
        pltpu.sync_copy(x_hbm.at[pl.ds(base, CHUNK)], idx_v)
        handles = []
        for j in range(NSTREAMS):
            handles.append(
                pltpu.async_copy(
                    lut_hbm.at[idx_v.at[pl.ds(j * IDX_SUB, IDX_SUB)]],
                    rows_v.at[pl.ds(j * IDX_SUB, IDX_SUB)],
                    gsem,
                )
            )
        for h in handles:
            h.wait()

        def scale_row(i, sc):
            for j in range(D_MODEL // 16):
                rows_v[i, pl.ds(j * 16, 16)] = (
                    rows_v[i, pl.ds(j * 16, 16)] * SCALE
                )
            return sc

        lax.fori_loop(0, CHUNK, scale_row, 0, unroll=4)
        pltpu.sync_copy(rows_v, out_hbm.at[pl.ds(base, CHUNK)])
        return carry

    lax.fori_loop(0, NCHUNKS, chunk_body, 0)


def kernel(x, lut):
    xf = x.reshape(-1).astype(jnp.int32)
    out = _embed_sc(xf, lut)
    return out.reshape(x.shape + (lut.shape[-1],))
